# baseline (speedup 1.0000x reference)
import functools

import jax
import jax.numpy as jnp
from jax import lax
from jax.experimental import pallas as pl
from jax.experimental.pallas import tpu as pltpu
from jax.experimental.pallas import tpu_sc as plsc

LANES = 16
N_WORKERS = 32
WAIT_LAG = 2


def _gather_el(*, n_attr, batch, d, v):
    b_per_w = batch // N_WORKERS
    per_w = b_per_w * n_attr

    mesh = plsc.VectorSubcoreMesh(core_axis_name="c", subcore_axis_name="s")

    @functools.partial(
        pl.kernel,
        mesh=mesh,
        compiler_params=pltpu.CompilerParams(use_tc_tiling_on_sc=False,
                                             needs_layout_passes=False),
        out_type=jax.ShapeDtypeStruct((n_attr * d, batch), jnp.float32),
        scratch_types=[
            pltpu.VMEM((per_w,), jnp.int32),
            pltpu.VMEM((WAIT_LAG, b_per_w), jnp.int32),
            pltpu.VMEM((n_attr * d, b_per_w), jnp.float32),
            pltpu.SemaphoreType.DMA,
            pltpu.SemaphoreType.DMA,
        ],
    )
    def k(x_hbm, wt_hbm, out_hbm, xv, vcols, blk, gsem, wsem):
        wid = lax.axis_index("s") * 2 + lax.axis_index("c")
        pltpu.sync_copy(x_hbm.at[pl.ds(wid * per_w, per_w)], xv)
        lane = lax.iota(jnp.int32, LANES) * n_attr

        pending = []
        for a in range(n_attr):
            vcol = vcols.at[a % WAIT_LAG]
            for c in range(b_per_w // LANES):
                vv = plsc.load_gather(xv, [lane + (c * LANES * n_attr + a)])
                vcol[pl.ds(c * LANES, LANES)] = vv
            batch_handles = []
            for dd in range(d):
                base = pl.multiple_of((a * d + dd) * v, 8)
                batch_handles.append(
                    pltpu.async_copy(wt_hbm.at[pl.ds(base, v)].at[vcol],
                                     blk.at[a * d + dd], gsem))
            pending.append(batch_handles)
            if len(pending) >= WAIT_LAG:
                for g in pending.pop(0):
                    g.wait()
        for batch_handles in pending:
            for g in batch_handles:
                g.wait()
        pltpu.async_copy(blk,
                         out_hbm.at[:, pl.ds(wid * b_per_w, b_per_w)],
                         wsem).wait()

    return k


def kernel(x, W):
    n_attr, v, d = W.shape
    x = x[:, x.shape[1] - n_attr:]
    batch = x.shape[0]
    xf = x.reshape(batch * n_attr).astype(jnp.int32)
    wt = jnp.transpose(W, (0, 2, 1)).reshape(n_attr * d * v)
    out_t = _gather_el(n_attr=n_attr, batch=batch, d=d, v=v)(xf, wt)
    return out_t.T.reshape(batch, n_attr * d)
